# Initial kernel scaffold; baseline (speedup 1.0000x reference)
import functools
import math

import jax
import jax.numpy as jnp
from jax import lax
from jax.experimental import pallas as pl
from jax.experimental.pallas import tpu as pltpu
from jax.experimental.pallas import tpu_sc as plsc

D = 1024
K = 8
DH = 512
DP = 128
B = 2
L = 2048
M = 64
A = 20
NU = 1024
MT = B * M
NR = MT * A
NW = 32
ROWS_W = NR // NW
OUT_W = (B * L) // NW


def _sc_gather(hL_flat, m_flat):
    mesh = plsc.VectorSubcoreMesh(core_axis_name="c", subcore_axis_name="s")

    @functools.partial(
        pl.kernel,
        out_type=(jax.ShapeDtypeStruct((NR, D), jnp.float32),
                  jax.ShapeDtypeStruct((MT, D), jnp.float32)),
        mesh=mesh,
        scratch_types=[
            pltpu.VMEM((MT,), jnp.int32),
            pltpu.VMEM((ROWS_W,), jnp.int32),
            pltpu.VMEM((ROWS_W, D), jnp.float32),
            pltpu.VMEM((16,), jnp.int32),
            pltpu.VMEM((16, D), jnp.float32),
            pltpu.SemaphoreType.DMA,
        ],
    )
    def gather_k(hL_hbm, midx_hbm, ha_hbm, hm_hbm,
                 midx_v, aidx_v, arows_v, midx2_v, mrows_v, sem):
        wid = lax.axis_index("s") * 2 + lax.axis_index("c")
        pltpu.sync_copy(midx_hbm, midx_v)
        base = wid * ROWS_W
        for cch in range(ROWS_W // 16):
            r16 = lax.iota(jnp.int32, 16) + (base + cch * 16)
            b16 = r16 // (M * A)
            mm = (r16 // A) % M
            a16 = r16 % A
            off = jnp.where(a16 < A // 2, a16 - A // 2, a16 - A // 2 + 1)
            mv = plsc.load_gather(midx_v, [b16 * M + mm])
            cand = jnp.clip(mv + off, 0, L - 1)
            aidx_v[pl.ds(cch * 16, 16)] = b16 * L + cand
        pltpu.async_copy(hL_hbm.at[aidx_v], arows_v, sem).wait()
        pltpu.sync_copy(arows_v, ha_hbm.at[pl.ds(base, ROWS_W)])

        @pl.when(wid < MT // 16)
        def _():
            j16 = lax.iota(jnp.int32, 16) + wid * 16
            mv = plsc.load_gather(midx_v, [j16])
            midx2_v[...] = (j16 // M) * L + mv
            pltpu.async_copy(hL_hbm.at[midx2_v], mrows_v, sem).wait()
            pltpu.sync_copy(mrows_v, hm_hbm.at[pl.ds(wid * 16, 16)])

    return gather_k(hL_flat, m_flat)


def _gelu(x):
    return 0.5 * x * (1.0 + lax.erf(x * (1.0 / math.sqrt(2.0))))


def _tc_body(ha_ref, hm_ref, w1a_ref, w1m_ref, w2_ref, b1_ref, b2_ref,
             wr_ref, br_ref, wq_ref, bq_ref, wk_ref, bk_ref,
             r_ref, rt_ref, mf_ref, off_ref, offok_ref, u_ref,
             out_ref, pair_ref, rw_ref):
    kk_id = pl.program_id(0)
    f32 = jnp.float32
    ha = ha_ref[...]
    hm = hm_ref[...]
    rmat = r_ref[...]

    @pl.when(kk_id == 0)
    def _():
        logits = jnp.dot(hm, wr_ref[...], preferred_element_type=f32) + br_ref[...]
        mx = jnp.max(logits, axis=1, keepdims=True)
        ex = jnp.exp(logits - mx)
        wts = ex / jnp.sum(ex, axis=1, keepdims=True)
        rw_ref[...] = jnp.dot(rmat, wts, preferred_element_type=f32)

    x1m = jnp.dot(hm, w1m_ref[0], preferred_element_type=f32)
    x1 = (jnp.dot(ha, w1a_ref[0], preferred_element_type=f32)
          + jnp.dot(rmat, x1m, preferred_element_type=f32) + b1_ref[0])
    x2 = jnp.dot(_gelu(x1), w2_ref[0], preferred_element_type=f32) + b2_ref[0]
    sel = (lax.broadcasted_iota(jnp.int32, (1, K), 1) == kk_id).astype(f32)
    wgt = jnp.sum(rw_ref[...] * sel, axis=1, keepdims=True)
    contrib = wgt * x2

    @pl.when(kk_id == 0)
    def _():
        pair_ref[...] = contrib

    @pl.when(kk_id > 0)
    def _():
        pair_ref[...] += contrib

    @pl.when(kk_id == K - 1)
    def _():
        q = jnp.dot(hm, wq_ref[...], preferred_element_type=f32) + bq_ref[...]
        kmat = jnp.dot(ha, wk_ref[...], preferred_element_type=f32) + bk_ref[...]
        qr = jnp.dot(rmat, q, preferred_element_type=f32)
        scores = jnp.sum(qr * kmat, axis=1, keepdims=True) * (1.0 / math.sqrt(DP))
        mrow = jnp.dot(rmat, mf_ref[...], preferred_element_type=f32)
        cand = mrow + off_ref[...]
        inr = (cand >= 0.0) & (cand <= L - 1.0) & (offok_ref[...] > 0.0)
        candc = jnp.clip(cand, 0.0, L - 1.0)
        ti = 256
        cnt_parts = []
        for t in range(NR // ti):
            u_sel = u_ref[0:1, :] if t < (M * A) // ti else u_ref[1:2, :]
            eqs = (candc[t * ti:(t + 1) * ti, :] == u_sel).astype(f32)
            cnt_parts.append(jnp.sum(eqs, axis=1, keepdims=True))
        cnt = jnp.concatenate(cnt_parts, axis=0)
        cnt = jnp.where(inr, cnt, 0.0)
        valid = cnt > 0.0
        neg = jnp.where(valid, scores, -1e30)
        gmax = jnp.max(neg)
        e = jnp.where(valid, cnt * jnp.exp(jnp.minimum(scores - gmax, 0.0)), 0.0)
        denom = jnp.dot(rt_ref[...], e, preferred_element_type=f32)
        denr = jnp.dot(rmat, denom, preferred_element_type=f32)
        w = e / jnp.maximum(denr, 1e-30)
        out_ref[...] = jnp.dot(rt_ref[...], w * pair_ref[...],
                               preferred_element_type=f32)


def _tc_specs():
    c = lambda i, j: (lambda k: (i, j))
    kb = lambda: (lambda k: (k, 0, 0))
    in_specs = [
        pl.BlockSpec((NR, D), c(0, 0)),
        pl.BlockSpec((MT, D), c(0, 0)),
        pl.BlockSpec((1, D, DH), kb()),
        pl.BlockSpec((1, D, DH), kb()),
        pl.BlockSpec((1, DH, D), kb()),
        pl.BlockSpec((1, 1, DH), kb()),
        pl.BlockSpec((1, 1, D), kb()),
        pl.BlockSpec((D, K), c(0, 0)),
        pl.BlockSpec((1, K), c(0, 0)),
        pl.BlockSpec((D, DP), c(0, 0)),
        pl.BlockSpec((1, DP), c(0, 0)),
        pl.BlockSpec((D, DP), c(0, 0)),
        pl.BlockSpec((1, DP), c(0, 0)),
        pl.BlockSpec((NR, MT), c(0, 0)),
        pl.BlockSpec((MT, NR), c(0, 0)),
        pl.BlockSpec((MT, 1), c(0, 0)),
        pl.BlockSpec((NR, 1), c(0, 0)),
        pl.BlockSpec((NR, 1), c(0, 0)),
        pl.BlockSpec((B, NU), c(0, 0)),
    ]
    out_spec = pl.BlockSpec((MT, D), c(0, 0))
    scratch = [pltpu.VMEM((NR, D), jnp.float32),
               pltpu.VMEM((NR, K), jnp.float32)]
    return in_specs, out_spec, scratch


def _tc_delta(*args):
    in_specs, out_spec, scratch = _tc_specs()
    return pl.pallas_call(
        _tc_body,
        grid=(K,),
        in_specs=in_specs,
        out_specs=out_spec,
        out_shape=jax.ShapeDtypeStruct((MT, D), jnp.float32),
        scratch_shapes=scratch,
        compiler_params=pltpu.CompilerParams(
            dimension_semantics=("arbitrary",)),
    )(*args)


def _sc_scatter(delta, m_flat):
    mesh = plsc.VectorSubcoreMesh(core_axis_name="c", subcore_axis_name="s")

    @functools.partial(
        pl.kernel,
        out_type=jax.ShapeDtypeStruct((B * L, D), jnp.float32),
        mesh=mesh,
        scratch_types=[
            pltpu.VMEM((16, D), jnp.float32),
            pltpu.VMEM((MT,), jnp.int32),
            pltpu.VMEM((1, D), jnp.float32),
        ],
    )
    def scatter_k(delta_hbm, midx_hbm, out_hbm, zbuf, midx_v, rowbuf):
        wid = lax.axis_index("s") * 2 + lax.axis_index("c")
        lo = wid * OUT_W
        pltpu.sync_copy(midx_hbm, midx_v)
        zero16 = jnp.zeros((16,), jnp.float32)
        for i in range(16):
            for c2 in range(D // 16):
                zbuf[i, pl.ds(c2 * 16, 16)] = zero16
        for j in range(OUT_W // 16):
            pltpu.sync_copy(zbuf, out_hbm.at[pl.ds(lo + j * 16, 16)])

        def body(m, carry):
            t = midx_v[m] + (m // M) * L

            @pl.when((t >= lo) & (t < lo + OUT_W))
            def _():
                pltpu.sync_copy(delta_hbm.at[pl.ds(m, 1)], rowbuf)
                pltpu.sync_copy(rowbuf, out_hbm.at[pl.ds(t, 1)])
            return carry

        lax.fori_loop(0, MT, body, 0)

    return scatter_k(delta, m_flat)


def kernel(h_L, mask_indices, unmasked_indices, range_r,
           Wr, br, W1, b1, W2, b2, Wq, bq, Wk, bk):
    hL_flat = h_L.reshape(B * L, D)
    m_flat = mask_indices.reshape(MT).astype(jnp.int32)
    ha, hm = _sc_gather(hL_flat, m_flat)

    rows = jnp.arange(NR)
    rmat = (rows[:, None] // A == jnp.arange(MT)[None, :]).astype(jnp.float32)
    offsets = jnp.concatenate([jnp.arange(-(A // 2), 0),
                               jnp.arange(1, A // 2 + 1)])
    off_flat = jnp.tile(offsets, MT).astype(jnp.float32)[:, None]
    offok = (jnp.abs(jnp.tile(offsets, MT)) <= range_r).astype(jnp.float32)[:, None]
    mf = m_flat.astype(jnp.float32)[:, None]
    u_f = unmasked_indices.astype(jnp.float32)

    delta = _tc_delta(
        ha, hm, W1[:, :D, :], W1[:, D:, :], W2,
        b1[:, None, :], b2[:, None, :], Wr, br[None, :],
        Wq, bq[None, :], Wk, bk[None, :],
        rmat, rmat.T, mf, off_flat, offok, u_f)

    out = _sc_scatter(delta, m_flat)
    return out.reshape(B, L, D)


# speedup vs baseline: 2.8717x; 2.8717x over previous
import functools
import math

import jax
import jax.numpy as jnp
from jax import lax
from jax.experimental import pallas as pl
from jax.experimental.pallas import tpu as pltpu
from jax.experimental.pallas import tpu_sc as plsc

D = 1024
K = 8
DH = 512
DP = 128
B = 2
L = 2048
M = 64
A = 20
NU = 1024
MT = B * M
NR = MT * A
NW = 32
ROWS_W = NR // NW
OUT_W = (B * L) // NW


def _sc_gather(hL_flat, aidx, midx_g):
    mesh = plsc.VectorSubcoreMesh(core_axis_name="c", subcore_axis_name="s")

    @functools.partial(
        pl.kernel,
        out_type=(jax.ShapeDtypeStruct((NR, D), jnp.float32),
                  jax.ShapeDtypeStruct((MT, D), jnp.float32)),
        mesh=mesh,
        scratch_types=[
            pltpu.VMEM((ROWS_W,), jnp.int32),
            pltpu.VMEM((ROWS_W, D), jnp.float32),
            pltpu.VMEM((16,), jnp.int32),
            pltpu.VMEM((16, D), jnp.float32),
            pltpu.SemaphoreType.DMA,
        ],
    )
    def gather_k(hL_hbm, aidx_hbm, midx_hbm, ha_hbm, hm_hbm,
                 aidx_v, arows_v, midx2_v, mrows_v, sem):
        wid = lax.axis_index("s") * 2 + lax.axis_index("c")
        base = wid * ROWS_W
        pltpu.sync_copy(aidx_hbm.at[pl.ds(base, ROWS_W)], aidx_v)
        pltpu.async_copy(hL_hbm.at[aidx_v], arows_v, sem).wait()
        pltpu.sync_copy(arows_v, ha_hbm.at[pl.ds(base, ROWS_W)])

        @pl.when(wid < MT // 16)
        def _():
            pltpu.sync_copy(midx_hbm.at[pl.ds(wid * 16, 16)], midx2_v)
            pltpu.async_copy(hL_hbm.at[midx2_v], mrows_v, sem).wait()
            pltpu.sync_copy(mrows_v, hm_hbm.at[pl.ds(wid * 16, 16)])

    return gather_k(hL_flat, aidx, midx_g)


def _gelu(x):
    return 0.5 * x * (1.0 + lax.erf(x * (1.0 / math.sqrt(2.0))))


NRH = NR // B
MH = MT // B


def _tc_body(ha_ref, hm_ref, w1_ref, w2_ref, b1_ref, b2_ref,
             wr_ref, br_ref, wq_ref, bq_ref, wk_ref, bk_ref,
             r_ref, rt_ref, cand_ref, inr_ref, u_ref,
             out_ref, pair_ref, rw_ref, habf_ref):
    kk_id = pl.program_id(1)
    f32 = jnp.float32
    bf16 = jnp.bfloat16
    hm = hm_ref[...].astype(bf16)
    rmat = r_ref[0].astype(bf16)

    @pl.when(kk_id == 0)
    def _():
        habf_ref[...] = ha_ref[...].astype(bf16)
        logits = jnp.dot(hm, wr_ref[...].astype(bf16),
                         preferred_element_type=f32) + br_ref[...]
        mx = jnp.max(logits, axis=1, keepdims=True)
        ex = jnp.exp(logits - mx)
        wts = ex / jnp.sum(ex, axis=1, keepdims=True)
        rw_ref[...] = jnp.dot(rmat, wts.astype(bf16), preferred_element_type=f32)

    w1 = w1_ref[0].astype(bf16)
    x1m = jnp.dot(hm, w1[D:], preferred_element_type=f32)
    x1 = (jnp.dot(habf_ref[...], w1[:D], preferred_element_type=f32)
          + jnp.dot(rmat, x1m.astype(bf16), preferred_element_type=f32)
          + b1_ref[0])
    sel = (lax.broadcasted_iota(jnp.int32, (1, K), 1) == kk_id).astype(f32)
    wgt = jnp.sum(rw_ref[...] * sel, axis=1, keepdims=True)
    gs = (wgt * _gelu(x1)).astype(bf16)
    contrib = jnp.dot(gs, w2_ref[0].astype(bf16), preferred_element_type=f32)

    @pl.when(kk_id == 0)
    def _():
        pair_ref[...] = contrib

    @pl.when(kk_id > 0)
    def _():
        pair_ref[...] += contrib

    @pl.when(kk_id == K - 1)
    def _():
        q = jnp.dot(hm_ref[...], wq_ref[...], preferred_element_type=f32) + bq_ref[...]
        kmat = jnp.dot(ha_ref[...], wk_ref[...], preferred_element_type=f32) + bk_ref[...]
        qr = jnp.dot(r_ref[0], q, preferred_element_type=f32)
        scores = jnp.sum(qr * kmat, axis=1, keepdims=True) * (1.0 / math.sqrt(DP))
        candc = cand_ref[0]
        inr = inr_ref[0] > 0.0
        u_sel = u_ref[0]
        ti = 256
        cnt_parts = []
        for t in range(NRH // ti):
            eqs = (candc[t * ti:(t + 1) * ti, :] == u_sel).astype(f32)
            cnt_parts.append(jnp.sum(eqs, axis=1, keepdims=True))
        cnt = jnp.concatenate(cnt_parts, axis=0)
        cnt = jnp.where(inr, cnt, 0.0)
        valid = cnt > 0.0
        neg = jnp.where(valid, scores, -1e30)
        gmax = jnp.max(neg)
        e = jnp.where(valid, cnt * jnp.exp(jnp.minimum(scores - gmax, 0.0)), 0.0)
        denom = jnp.dot(rt_ref[0], e, preferred_element_type=f32)
        denr = jnp.dot(r_ref[0], denom, preferred_element_type=f32)
        w = e / jnp.maximum(denr, 1e-30)
        pairtot = pair_ref[...] + jnp.dot(rw_ref[...], b2_ref[...],
                                          preferred_element_type=f32)
        out_ref[...] = jnp.dot(rt_ref[0], w * pairtot,
                               preferred_element_type=f32)


def _tc_specs():
    c = lambda i, j: (lambda r, k: (i, j))
    rb = lambda: (lambda r, k: (r, 0))
    kb = lambda: (lambda r, k: (k, 0, 0))
    in_specs = [
        pl.BlockSpec((NRH, D), rb()),
        pl.BlockSpec((MH, D), rb()),
        pl.BlockSpec((1, 2 * D, DH), kb()),
        pl.BlockSpec((1, DH, D), kb()),
        pl.BlockSpec((1, 1, DH), kb()),
        pl.BlockSpec((K, D), c(0, 0)),
        pl.BlockSpec((D, K), c(0, 0)),
        pl.BlockSpec((1, K), c(0, 0)),
        pl.BlockSpec((D, DP), c(0, 0)),
        pl.BlockSpec((1, DP), c(0, 0)),
        pl.BlockSpec((D, DP), c(0, 0)),
        pl.BlockSpec((1, DP), c(0, 0)),
        pl.BlockSpec((1, NRH, MH), lambda r, k: (0, 0, 0)),
        pl.BlockSpec((1, MH, NRH), lambda r, k: (0, 0, 0)),
        pl.BlockSpec((1, NRH, 1), lambda r, k: (r, 0, 0)),
        pl.BlockSpec((1, NRH, 1), lambda r, k: (r, 0, 0)),
        pl.BlockSpec((1, 1, NU), lambda r, k: (r, 0, 0)),
    ]
    out_spec = pl.BlockSpec((MH, D), rb())
    scratch = [pltpu.VMEM((NRH, D), jnp.float32),
               pltpu.VMEM((NRH, K), jnp.float32),
               pltpu.VMEM((NRH, D), jnp.bfloat16)]
    return in_specs, out_spec, scratch


def _tc_delta(*args):
    in_specs, out_spec, scratch = _tc_specs()
    return pl.pallas_call(
        _tc_body,
        grid=(B, K),
        in_specs=in_specs,
        out_specs=out_spec,
        out_shape=jax.ShapeDtypeStruct((MT, D), jnp.float32),
        scratch_shapes=scratch,
        compiler_params=pltpu.CompilerParams(
            dimension_semantics=("arbitrary", "arbitrary")),
    )(*args)


def _sc_scatter(delta, m_flat):
    mesh = plsc.VectorSubcoreMesh(core_axis_name="c", subcore_axis_name="s")

    @functools.partial(
        pl.kernel,
        out_type=jax.ShapeDtypeStruct((B * L, D), jnp.float32),
        mesh=mesh,
        scratch_types=[
            pltpu.VMEM((16, D), jnp.float32),
            pltpu.VMEM((MT,), jnp.int32),
            pltpu.VMEM((1, D), jnp.float32),
        ],
    )
    def scatter_k(delta_hbm, midx_hbm, out_hbm, zbuf, midx_v, rowbuf):
        wid = lax.axis_index("s") * 2 + lax.axis_index("c")
        lo = wid * OUT_W
        pltpu.sync_copy(midx_hbm, midx_v)
        zero16 = jnp.zeros((16,), jnp.float32)
        for i in range(16):
            for c2 in range(D // 16):
                zbuf[i, pl.ds(c2 * 16, 16)] = zero16
        for j in range(OUT_W // 16):
            pltpu.sync_copy(zbuf, out_hbm.at[pl.ds(lo + j * 16, 16)])

        def body(ch, carry):
            v = midx_v[pl.ds(ch * 16, 16)]
            mbase = ch * 16
            for m0 in range(16):
                m = mbase + m0
                t = v[m0] + (m // M) * L

                @pl.when((t >= lo) & (t < lo + OUT_W))
                def _(t=t, m=m):
                    pltpu.sync_copy(delta_hbm.at[pl.ds(m, 1)], rowbuf)
                    pltpu.sync_copy(rowbuf, out_hbm.at[pl.ds(t, 1)])
            return carry

        lax.fori_loop(0, MT // 16, body, 0)

    return scatter_k(delta, m_flat)


def kernel(h_L, mask_indices, unmasked_indices, range_r,
           Wr, br, W1, b1, W2, b2, Wq, bq, Wk, bk):
    hL_flat = h_L.reshape(B * L, D)
    m_flat = mask_indices.reshape(MT).astype(jnp.int32)
    offs_i = jnp.concatenate([jnp.arange(-(A // 2), 0),
                              jnp.arange(1, A // 2 + 1)]).astype(jnp.int32)
    cand_i = jnp.clip(mask_indices.astype(jnp.int32)[:, :, None] + offs_i, 0, L - 1)
    aidx = (cand_i + (jnp.arange(B, dtype=jnp.int32) * L)[:, None, None]).reshape(NR)
    midx_g = (mask_indices.astype(jnp.int32)
              + (jnp.arange(B, dtype=jnp.int32) * L)[:, None]).reshape(MT)
    ha, hm = _sc_gather(hL_flat, aidx, midx_g)

    rowsh = jnp.arange(NRH)
    rh = (rowsh[:, None] // A == jnp.arange(MH)[None, :]).astype(jnp.float32)
    cand_raw = mask_indices.astype(jnp.int32)[:, :, None] + offs_i
    inr_i = ((cand_raw >= 0) & (cand_raw < L)
             & (jnp.abs(offs_i) <= range_r)[None, None, :])
    candf = cand_i.astype(jnp.float32).reshape(B, NRH, 1)
    inrf = inr_i.astype(jnp.float32).reshape(B, NRH, 1)
    u_f = unmasked_indices.astype(jnp.float32)[:, None, :]

    delta = _tc_delta(
        ha, hm, W1, W2,
        b1[:, None, :], b2, Wr, br[None, :],
        Wq, bq[None, :], Wk, bk[None, :],
        rh[None], rh.T[None], candf, inrf, u_f)

    out = _sc_scatter(delta, m_flat)
    return out.reshape(B, L, D)
